# baseline (speedup 1.0000x reference)
import math

import jax
import jax.numpy as jnp
from jax import lax
from jax.experimental import pallas as pl

_N_BLOCK = 400
_E_BLOCK = 6400
_TIME_HALF = 16
_LOG_MAX_POS = math.log(10000.0)


def _fused_kernel(h_ref, t_ref, f_ref, attr_ref, atom_ref,
                  w_emb1_ref, w_emb2_ref, b_emb_ref,
                  w_in1_ref, w_in2_ref, b_in_ref,
                  freq_ref, phase_ref, table_rep_t_ref,
                  node_out_ref, edge_out_ref):
    bn = h_ref.shape[1]
    hv = h_ref[0]
    tv = t_ref[0].astype(jnp.float32)

    be = attr_ref.shape[2]
    av = attr_ref[0]
    e_iota = lax.broadcasted_iota(jnp.int32, (50, be), 0)
    e_onehot = (e_iota == av).astype(jnp.float32)
    e2d = jnp.dot(table_rep_t_ref[...], e_onehot,
                  preferred_element_type=jnp.float32)
    edge_out_ref[...] = e2d.reshape(edge_out_ref.shape)

    atom_iota = lax.broadcasted_iota(jnp.int32, (bn, 100), 1)
    onehot = (hv == atom_iota).astype(jnp.float32)
    atom_embed = jnp.dot(onehot, atom_ref[...],
                         preferred_element_type=jnp.float32)

    h_feat = (jnp.dot(atom_embed, w_emb1_ref[...],
                      preferred_element_type=jnp.float32)
              + jnp.dot(f_ref[...], w_emb2_ref[...],
                        preferred_element_type=jnp.float32)
              + b_emb_ref[...])

    arg = tv * freq_ref[...] + phase_ref[...]
    t_emb = jnp.sin(arg)

    res = (jnp.dot(h_feat, w_in1_ref[...],
                   preferred_element_type=jnp.float32)
           + jnp.dot(t_emb, w_in2_ref[...],
                     preferred_element_type=jnp.float32)
           + b_in_ref[...])
    node_out_ref[...] = jnp.broadcast_to(res[:, None, :], node_out_ref.shape)


def kernel(diffusion_t, x, h, f, edge_index, edge_attr, batch, atom_table,
           W_emb, b_emb, edge_table, cond_table, W_in, b_in):
    N, FT = f.shape
    E = edge_attr.shape[0]
    T = x.shape[-1]
    HID = W_in.shape[0]
    ED = edge_table.shape[1]
    NODE = atom_table.shape[1]

    nb = N // _N_BLOCK

    w_emb_t = W_emb.T
    w_emb1 = w_emb_t[:NODE, :]
    w_emb2 = w_emb_t[NODE:, :]
    w_in_t = W_in.T
    w_in1 = w_in_t[:NODE, :]
    w_in2 = w_in_t[NODE:, :]
    b_in_row = b_in[None, :]
    b_emb_row = b_emb[None, :]
    table_rep_t = jnp.repeat(edge_table.T, T, axis=0)

    half = _TIME_HALF
    j = jnp.arange(half, dtype=jnp.float32)
    freq_half = jnp.exp(j * (-_LOG_MAX_POS / (half - 1)))
    freq_row = jnp.concatenate([freq_half, freq_half])[None, :]
    phase_row = jnp.concatenate([jnp.zeros(half),
                                 jnp.full((half,), math.pi / 2)]).astype(
                                     jnp.float32)[None, :]

    h3 = h.astype(jnp.int32).reshape(nb, _N_BLOCK, 1)
    t3 = diffusion_t.astype(jnp.int32).reshape(nb, _N_BLOCK, 1)
    a3 = edge_attr.astype(jnp.int32).reshape(nb, 1, _E_BLOCK)

    h3d, e3d = pl.pallas_call(
        _fused_kernel,
        grid=(nb,),
        in_specs=[
            pl.BlockSpec((1, _N_BLOCK, 1), lambda i: (i, 0, 0)),
            pl.BlockSpec((1, _N_BLOCK, 1), lambda i: (i, 0, 0)),
            pl.BlockSpec((_N_BLOCK, FT), lambda i: (i, 0)),
            pl.BlockSpec((1, 1, _E_BLOCK), lambda i: (i, 0, 0)),
            pl.BlockSpec(atom_table.shape, lambda i: (0, 0)),
            pl.BlockSpec(w_emb1.shape, lambda i: (0, 0)),
            pl.BlockSpec(w_emb2.shape, lambda i: (0, 0)),
            pl.BlockSpec(b_emb_row.shape, lambda i: (0, 0)),
            pl.BlockSpec(w_in1.shape, lambda i: (0, 0)),
            pl.BlockSpec(w_in2.shape, lambda i: (0, 0)),
            pl.BlockSpec(b_in_row.shape, lambda i: (0, 0)),
            pl.BlockSpec(freq_row.shape, lambda i: (0, 0)),
            pl.BlockSpec(phase_row.shape, lambda i: (0, 0)),
            pl.BlockSpec(table_rep_t.shape, lambda i: (0, 0)),
        ],
        out_specs=[
            pl.BlockSpec((_N_BLOCK, T, HID), lambda i: (i, 0, 0)),
            pl.BlockSpec((ED, T, _E_BLOCK), lambda i: (0, 0, i)),
        ],
        out_shape=[
            jax.ShapeDtypeStruct((N, T, HID), jnp.float32),
            jax.ShapeDtypeStruct((ED, T, E), jnp.float32),
        ],
    )(h3, t3, f, a3, atom_table, w_emb1, w_emb2, b_emb_row,
      w_in1, w_in2, b_in_row, freq_row, phase_row, table_rep_t)

    h_out = jnp.transpose(h3d, (0, 2, 1))
    edge_out = jnp.transpose(e3d, (2, 0, 1))
    return (x, h_out, edge_out)


# speedup vs baseline: 1.1005x; 1.1005x over previous
import math

import jax
import jax.numpy as jnp
from jax import lax
from jax.experimental import pallas as pl

_N_BLOCK = 1000
_E_BLOCK = 16000
_TIME_HALF = 16
_LOG_MAX_POS = math.log(10000.0)


def _fused_kernel(h_ref, t_ref, f_ref, attr_ref, atom_ref,
                  w_emb1_ref, w_emb2_ref, b_emb_ref,
                  w_in1_ref, w_in2_ref, b_in_ref,
                  freq_ref, phase_ref, table_rep_t_ref,
                  node_out_ref, edge_out_ref):
    bn = h_ref.shape[1]
    hv = h_ref[0]
    tv = t_ref[0].astype(jnp.float32)

    be = attr_ref.shape[2]
    av = attr_ref[0]
    e_iota = lax.broadcasted_iota(jnp.int32, (50, be), 0)
    e_onehot = (e_iota == av).astype(jnp.float32)
    e2d = jnp.dot(table_rep_t_ref[...], e_onehot,
                  preferred_element_type=jnp.float32)
    edge_out_ref[...] = e2d.reshape(edge_out_ref.shape)

    atom_iota = lax.broadcasted_iota(jnp.int32, (bn, 100), 1)
    onehot = (hv == atom_iota).astype(jnp.float32)
    atom_embed = jnp.dot(onehot, atom_ref[...],
                         preferred_element_type=jnp.float32)

    h_feat = (jnp.dot(atom_embed, w_emb1_ref[...],
                      preferred_element_type=jnp.float32)
              + jnp.dot(f_ref[...], w_emb2_ref[...],
                        preferred_element_type=jnp.float32)
              + b_emb_ref[...])

    arg = tv * freq_ref[...] + phase_ref[...]
    t_emb = jnp.sin(arg)

    res = (jnp.dot(h_feat, w_in1_ref[...],
                   preferred_element_type=jnp.float32)
           + jnp.dot(t_emb, w_in2_ref[...],
                     preferred_element_type=jnp.float32)
           + b_in_ref[...])
    node_out_ref[...] = jnp.broadcast_to(res[:, None, :], node_out_ref.shape)


def kernel(diffusion_t, x, h, f, edge_index, edge_attr, batch, atom_table,
           W_emb, b_emb, edge_table, cond_table, W_in, b_in):
    N, FT = f.shape
    E = edge_attr.shape[0]
    T = x.shape[-1]
    HID = W_in.shape[0]
    ED = edge_table.shape[1]
    NODE = atom_table.shape[1]

    nb = N // _N_BLOCK

    w_emb_t = W_emb.T
    w_emb1 = w_emb_t[:NODE, :]
    w_emb2 = w_emb_t[NODE:, :]
    w_in_t = W_in.T
    w_in1 = w_in_t[:NODE, :]
    w_in2 = w_in_t[NODE:, :]
    b_in_row = b_in[None, :]
    b_emb_row = b_emb[None, :]
    table_rep_t = jnp.repeat(edge_table.T, T, axis=0)

    half = _TIME_HALF
    j = jnp.arange(half, dtype=jnp.float32)
    freq_half = jnp.exp(j * (-_LOG_MAX_POS / (half - 1)))
    freq_row = jnp.concatenate([freq_half, freq_half])[None, :]
    phase_row = jnp.concatenate([jnp.zeros(half),
                                 jnp.full((half,), math.pi / 2)]).astype(
                                     jnp.float32)[None, :]

    h3 = h.astype(jnp.int32).reshape(nb, _N_BLOCK, 1)
    t3 = diffusion_t.astype(jnp.int32).reshape(nb, _N_BLOCK, 1)
    a3 = edge_attr.astype(jnp.int32).reshape(nb, 1, _E_BLOCK)

    h3d, e3d = pl.pallas_call(
        _fused_kernel,
        grid=(nb,),
        in_specs=[
            pl.BlockSpec((1, _N_BLOCK, 1), lambda i: (i, 0, 0)),
            pl.BlockSpec((1, _N_BLOCK, 1), lambda i: (i, 0, 0)),
            pl.BlockSpec((_N_BLOCK, FT), lambda i: (i, 0)),
            pl.BlockSpec((1, 1, _E_BLOCK), lambda i: (i, 0, 0)),
            pl.BlockSpec(atom_table.shape, lambda i: (0, 0)),
            pl.BlockSpec(w_emb1.shape, lambda i: (0, 0)),
            pl.BlockSpec(w_emb2.shape, lambda i: (0, 0)),
            pl.BlockSpec(b_emb_row.shape, lambda i: (0, 0)),
            pl.BlockSpec(w_in1.shape, lambda i: (0, 0)),
            pl.BlockSpec(w_in2.shape, lambda i: (0, 0)),
            pl.BlockSpec(b_in_row.shape, lambda i: (0, 0)),
            pl.BlockSpec(freq_row.shape, lambda i: (0, 0)),
            pl.BlockSpec(phase_row.shape, lambda i: (0, 0)),
            pl.BlockSpec(table_rep_t.shape, lambda i: (0, 0)),
        ],
        out_specs=[
            pl.BlockSpec((_N_BLOCK, T, HID), lambda i: (i, 0, 0)),
            pl.BlockSpec((ED, T, _E_BLOCK), lambda i: (0, 0, i)),
        ],
        out_shape=[
            jax.ShapeDtypeStruct((N, T, HID), jnp.float32),
            jax.ShapeDtypeStruct((ED, T, E), jnp.float32),
        ],
    )(h3, t3, f, a3, atom_table, w_emb1, w_emb2, b_emb_row,
      w_in1, w_in2, b_in_row, freq_row, phase_row, table_rep_t)

    h_out = jnp.transpose(h3d, (0, 2, 1))
    edge_out = jnp.transpose(e3d, (2, 0, 1))
    return (x, h_out, edge_out)
